# baseline (speedup 1.0000x reference)
import functools
import jax
import jax.numpy as jnp
from jax import lax
from jax.experimental import pallas as pl
from jax.experimental.pallas import tpu as pltpu
from jax.experimental.pallas import tpu_sc as plsc

_K = 256
_B = 8
_C = 256
_V = 32768
_NW_PER_B = 4
_VPW = _K // _NW_PER_B
_NSLAB = _VPW // 16


def _score_body(w_ref, f_ref, s_ref):
    f = f_ref[0]
    w = w_ref[...]
    s_ref[0] = jnp.dot(w, f, preferred_element_type=jnp.float32)


def _select_body(s_ref, il_ref, ils_ref):
    s = s_ref[...]
    B, V = s.shape
    _INT_MIN = jnp.int32(-2147483648)
    ki = lax.bitcast_convert_type(s, jnp.int32)
    key = jnp.where(ki >= 0, ki, ki ^ jnp.int32(0x7FFFFFFF))
    x = key ^ _INT_MIN

    prefix = jnp.zeros((B, 1), jnp.int32)
    need = jnp.full((B, 1), _K, jnp.int32)
    for bit in range(31, -1, -1):
        b = jnp.int32(1 << bit) if bit < 31 else _INT_MIN
        lowmask = jnp.int32((1 << (bit + 1)) - 1) if bit < 31 else jnp.int32(-1)
        himask = ~lowmask
        cand_hi = ((x & himask) == prefix) & ((x & b) != 0)
        c1 = jnp.sum(cand_hi.astype(jnp.int32), axis=1, keepdims=True)
        go_hi = c1 >= need
        prefix = jnp.where(go_hi, prefix | b, prefix)
        need = jnp.where(go_hi, need, need - c1)
    t_key = prefix ^ _INT_MIN
    eq = key == t_key

    idx = lax.broadcasted_iota(jnp.int32, (B, V), 1)
    prefix2 = jnp.zeros((B, 1), jnp.int32)
    need2 = need
    for bit in range(14, -1, -1):
        b = jnp.int32(1 << bit)
        himask2 = ~jnp.int32((1 << (bit + 1)) - 1)
        cand_lo = eq & ((idx & himask2) == prefix2) & ((idx & b) == 0)
        c0 = jnp.sum(cand_lo.astype(jnp.int32), axis=1, keepdims=True)
        stay_lo = c0 >= need2
        prefix2 = jnp.where(stay_lo, prefix2, prefix2 | b)
        need2 = jnp.where(stay_lo, need2, need2 - c0)

    m = (key > t_key) | (eq & (idx <= prefix2))
    mi = m.astype(jnp.int32)

    inc = mi
    for sh in [1, 2, 4, 8, 16, 32, 64, 128, 256, 512, 1024, 2048, 4096,
               8192, 16384]:
        inc = inc + jnp.concatenate(
            [jnp.zeros((B, sh), jnp.int32), inc[:, :V - sh]], axis=1)
    rank = inc - mi

    rhi = rank >> 4
    rlo = rank & 15
    mf = m.astype(jnp.float32)
    vh = (idx >> 7).astype(jnp.float32) * mf
    vl = (idx & 127).astype(jnp.float32) * mf
    jot = lax.broadcasted_iota(jnp.int32, (16, V), 0)

    for bb in range(B):
        mrow = m[bb][None, :]
        ahi = jnp.where((jot == rhi[bb][None, :]) & mrow,
                        jnp.float32(1), jnp.float32(0))
        alo = jnp.where((jot == rlo[bb][None, :]) & mrow,
                        jnp.float32(1), jnp.float32(0))
        dn = (((1,), (1,)), ((), ()))
        oh = lax.dot_general(ahi * vh[bb][None, :], alo, dn,
                             preferred_element_type=jnp.float32)
        ol = lax.dot_general(ahi * vl[bb][None, :], alo, dn,
                             preferred_element_type=jnp.float32)
        ids = oh.astype(jnp.int32) * 128 + ol.astype(jnp.int32)
        il_ref[bb] = ids
        k16 = (lax.broadcasted_iota(jnp.int32, (_K, 16), 0) // 16 ==
               lax.broadcasted_iota(jnp.int32, (_K, 16), 1)).astype(jnp.float32)
        t1h = lax.dot_general(k16, oh, (((1,), (0,)), ((), ())),
                              preferred_element_type=jnp.float32)
        t1l = lax.dot_general(k16, ol, (((1,), (0,)), ((), ())),
                              preferred_element_type=jnp.float32)
        selm = (lax.broadcasted_iota(jnp.int32, (_K, 16), 0) % 16 ==
                lax.broadcasted_iota(jnp.int32, (_K, 16), 1)).astype(jnp.float32)
        sph = jnp.sum(t1h * selm, axis=1, keepdims=True)
        spl = jnp.sum(t1l * selm, axis=1, keepdims=True)
        sp = sph.astype(jnp.int32) * 128 + spl.astype(jnp.int32)
        ils_ref[bb] = jnp.broadcast_to(sp, (_K, 16))


def _isplat(x):
    return jnp.full((16,), x, jnp.int32)


def _sc_gather_body(fmap_ref, ils_ref, out_ref,
                    idlsp_ref, cpat_ref, gidx_ref, gout_ref,
                    accb_ref, didx_ref, stage_ref, acc_ref, sem):
    cid = lax.axis_index("c")
    sid = lax.axis_index("s")
    b_local = sid // _NW_PER_B
    b = cid * 4 + b_local
    q = sid % _NW_PER_B

    @pl.when(sid == 0)
    def _():
        def zloop(i, _):
            stage_ref[pl.ds(i * 16, 16)] = jnp.zeros((16,), jnp.float32)
            return 0
        lax.fori_loop(0, (4 * _C) // 16, zloop, 0)
        pltpu.sync_copy(stage_ref, acc_ref)
    plsc.subcore_barrier()

    pltpu.sync_copy(
        ils_ref.at[pl.ds((b * _K + q * _VPW) * 16, _VPW * 16)], idlsp_ref)

    lanes = lax.iota(jnp.int32, 16)
    fbase = b * (_C * _V)
    acc_row = b_local * _C
    for t in range(16):
        cpat_ref[pl.ds(t * 16, 16)] = (
            (lanes + _isplat(16 * t)) * _isplat(_V) + _isplat(fbase))
        didx_ref[pl.ds(t * 16, 16)] = lanes + _isplat(acc_row + 16 * t)
        accb_ref[pl.ds(t * 16, 16)] = jnp.zeros((16,), jnp.float32)

    def jloop(j, _):
        vsp = idlsp_ref[pl.ds(j * 16, 16)]
        base = j * _C
        for t in range(16):
            cp = cpat_ref[pl.ds(t * 16, 16)]
            gidx_ref[pl.ds(base + t * 16, 16)] = vsp + cp
        return 0
    lax.fori_loop(0, _VPW, jloop, 0)

    pltpu.async_copy(fmap_ref.at[gidx_ref], gout_ref, sem).wait()

    def aloop(j, _):
        base = j * _C
        for t in range(16):
            accb_ref[pl.ds(t * 16, 16)] = (
                accb_ref[pl.ds(t * 16, 16)] +
                gout_ref[pl.ds(base + t * 16, 16)])
        return 0
    lax.fori_loop(0, _VPW, aloop, 0)

    pltpu.sync_copy(accb_ref, acc_ref.at[didx_ref], add=True)

    plsc.subcore_barrier()

    @pl.when(sid == 0)
    def _():
        pltpu.sync_copy(acc_ref, stage_ref)

        def scale(i, _):
            stage_ref[pl.ds(i * 16, 16)] = (
                stage_ref[pl.ds(i * 16, 16)] * jnp.float32(1.0 / _K))
            return 0
        lax.fori_loop(0, (4 * _C) // 16, scale, 0)
        pltpu.sync_copy(stage_ref, out_ref.at[pl.ds(cid * 4 * _C, 4 * _C)])


def kernel(Fmap, score_w, score_b):
    B, C, D, H, W = Fmap.shape
    V = D * H * W
    f = Fmap.reshape(B, C, V)
    w = score_w.reshape(1, C)

    BV = 2048
    nj = V // BV

    s = pl.pallas_call(
        _score_body,
        grid=(B, nj),
        in_specs=[
            pl.BlockSpec((1, C), lambda b, j: (0, 0)),
            pl.BlockSpec((1, C, BV), lambda b, j: (b, 0, j)),
        ],
        out_specs=pl.BlockSpec((1, 1, BV), lambda b, j: (b * nj + j, 0, 0)),
        out_shape=jax.ShapeDtypeStruct((B * nj, 1, BV), jnp.float32),
    )(w, f)
    s = s.reshape(B, V)

    ilist, ilsplat = pl.pallas_call(
        _select_body,
        out_shape=[
            jax.ShapeDtypeStruct((B, 16, 16), jnp.int32),
            jax.ShapeDtypeStruct((B, _K, 16), jnp.int32),
        ],
    )(s)

    mesh = plsc.VectorSubcoreMesh(core_axis_name="c", subcore_axis_name="s")
    sc = functools.partial(
        pl.kernel, mesh=mesh,
        out_type=jax.ShapeDtypeStruct((B * C,), jnp.float32),
        scratch_types=[
            pltpu.VMEM((_VPW * 16,), jnp.int32),
            pltpu.VMEM((_C,), jnp.int32),
            pltpu.VMEM((_VPW * _C,), jnp.int32),
            pltpu.VMEM((_VPW * _C,), jnp.float32),
            pltpu.VMEM((_C,), jnp.float32),
            pltpu.VMEM((_C,), jnp.int32),
            pltpu.VMEM((4 * _C,), jnp.float32),
            pltpu.VMEM_SHARED((4 * _C,), jnp.float32),
            pltpu.SemaphoreType.DMA,
        ],
    )(_sc_gather_body)

    out = sc(f.reshape(B * C * V), ilsplat.reshape(B * _K * 16))
    return out.reshape(B, C)


# speedup vs baseline: 2.8738x; 2.8738x over previous
import functools
import jax
import jax.numpy as jnp
from jax.experimental import pallas as pl

_K = 256


def _score_body(w_ref, f_ref, s_ref):
    f = f_ref[0]
    w = w_ref[...]
    s_ref[0] = jnp.dot(w, f, preferred_element_type=jnp.float32)


def _select_body(s_ref, m_ref):
    s = s_ref[...]
    B, V = s.shape
    _INT_MIN = jnp.int32(-2147483648)
    ki = jax.lax.bitcast_convert_type(s, jnp.int32)
    key = jnp.where(ki >= 0, ki, ki ^ jnp.int32(0x7FFFFFFF))
    x = key ^ _INT_MIN

    prefix = jnp.zeros((B, 1), jnp.int32)
    need = jnp.full((B, 1), _K, jnp.int32)
    for bit in range(31, -1, -1):
        b = jnp.int32(1 << bit) if bit < 31 else _INT_MIN
        lowmask = jnp.int32((1 << (bit + 1)) - 1) if bit < 31 else jnp.int32(-1)
        himask = ~lowmask
        cand_hi = ((x & himask) == prefix) & ((x & b) != 0)
        c1 = jnp.sum(cand_hi.astype(jnp.int32), axis=1, keepdims=True)
        go_hi = c1 >= need
        prefix = jnp.where(go_hi, prefix | b, prefix)
        need = jnp.where(go_hi, need, need - c1)
    t_key = prefix ^ _INT_MIN
    gt = key > t_key
    eq = key == t_key

    idx = jax.lax.broadcasted_iota(jnp.int32, (B, V), 1)
    prefix2 = jnp.zeros((B, 1), jnp.int32)
    need2 = need
    for bit in range(14, -1, -1):
        b = jnp.int32(1 << bit)
        himask2 = ~jnp.int32((1 << (bit + 1)) - 1)
        cand_lo = eq & ((idx & himask2) == prefix2) & ((idx & b) == 0)
        c0 = jnp.sum(cand_lo.astype(jnp.int32), axis=1, keepdims=True)
        stay_lo = c0 >= need2
        prefix2 = jnp.where(stay_lo, prefix2, prefix2 | b)
        need2 = jnp.where(stay_lo, need2, need2 - c0)
    t2 = prefix2

    sel = gt | (eq & (idx <= t2))
    m_ref[...] = sel.astype(jnp.float32)


def _mean_body(f_ref, m_ref, o_ref, *, nj):
    j = pl.program_id(1)

    @pl.when(j == 0)
    def _():
        o_ref[...] = jnp.zeros_like(o_ref)

    f = f_ref[0]
    m = m_ref[0]
    o_ref[0] += jax.lax.dot_general(
        m, f, (((1,), (1,)), ((), ())),
        preferred_element_type=jnp.float32)

    @pl.when(j == nj - 1)
    def _():
        o_ref[0] *= jnp.float32(1.0 / _K)


def kernel(Fmap, score_w, score_b):
    B, C, D, H, W = Fmap.shape
    V = D * H * W
    f = Fmap.reshape(B, C, V)
    w = score_w.reshape(1, C)

    BV = 4096
    nj = V // BV

    s = pl.pallas_call(
        _score_body,
        grid=(B, nj),
        in_specs=[
            pl.BlockSpec((1, C), lambda b, j: (0, 0)),
            pl.BlockSpec((1, C, BV), lambda b, j: (b, 0, j)),
        ],
        out_specs=pl.BlockSpec((1, 1, BV), lambda b, j: (b * nj + j, 0, 0)),
        out_shape=jax.ShapeDtypeStruct((B * nj, 1, BV), jnp.float32),
    )(w, f)

    mask = pl.pallas_call(
        _select_body,
        out_shape=jax.ShapeDtypeStruct((B, V), jnp.float32),
    )(s.reshape(B, V))

    out = pl.pallas_call(
        functools.partial(_mean_body, nj=nj),
        grid=(B, nj),
        in_specs=[
            pl.BlockSpec((1, C, BV), lambda b, j: (b, 0, j)),
            pl.BlockSpec((1, 1, BV), lambda b, j: (b * nj + j, 0, 0)),
        ],
        out_specs=pl.BlockSpec((1, 1, C), lambda b, j: (b, 0, 0)),
        out_shape=jax.ShapeDtypeStruct((B, 1, C), jnp.float32),
    )(f, mask.reshape(B * nj, 1, BV))

    return out.reshape(B, C)
